# baseline (speedup 1.0000x reference)
import functools

import jax
import jax.numpy as jnp
from jax.experimental import pallas as pl
from jax.experimental.pallas import tpu as pltpu
from jax.experimental.pallas import tpu_sc as plsc

K = 10
NC, NS = 2, 16
NW = NC * NS
CHUNK = 128
VOCAB_ = 400000
DIM_ = 300
BLOCK = 3200
F_TC = 192000
ROWS_SC = VOCAB_ - F_TC


def _sc_score_body(w_hbm, wv_hbm, num_hbm, sq_hbm, wbuf, buf, nbuf, sbuf, sems):
    nchunks = ROWS_SC // CHUNK
    wid = jax.lax.axis_index("s") * NC + jax.lax.axis_index("c")
    nt = (nchunks - 1 - wid) // NW + 1

    pltpu.sync_copy(w_hbm, wbuf)
    lane = jax.lax.iota(jnp.int32, 16)
    m01 = jnp.where(lane >= 4, jnp.float32(1.0), jnp.float32(0.0))
    def _rb(v):
        c = v * jnp.float32(65537.0)
        return c - (c - v)

    wjs = [_rb(wbuf[pl.ds(16 * j, 16)]) for j in range(18)]
    wt = _rb(wbuf[pl.ds(284, 16)]) * m01

    def copy_in(t, par):
        g = wid + NW * t
        return pltpu.make_async_copy(
            wv_hbm.at[pl.ds(F_TC + g * CHUNK, CHUNK), :],
            buf.at[par], sems.at[par])

    copy_in(0, 0).start()

    def chunk_body(t, carry):
        par = jax.lax.rem(t, 2)
        g = wid + NW * t
        copy_in(t, par).wait()

        @pl.when(t + 1 < nt)
        def _():
            copy_in(t + 1, 1 - par).start()

        @plsc.parallel_loop(0, CHUNK, unroll=2)
        def _row(r):
            x = buf[par, r, pl.ds(0, 16)]
            acc_n = _rb(x) * wjs[0]
            acc_s = x * x
            for j in range(1, 18):
                x = buf[par, r, pl.ds(16 * j, 16)]
                acc_n = acc_n + _rb(x) * wjs[j]
                acc_s = acc_s + x * x
            x = buf[par, r, pl.ds(284, 16)]
            acc_n = acc_n + _rb(x) * wt
            xm = x * m01
            acc_s = acc_s + xm * xm
            pvec = jnp.full((16,), par, jnp.int32)
            rvec = jnp.full((16,), r, jnp.int32)
            lane0 = lane == 0
            plsc.store_scatter(nbuf, [pvec, rvec],
                               jnp.full((16,), jnp.sum(acc_n), jnp.float32),
                               mask=lane0)
            plsc.store_scatter(sbuf, [pvec, rvec],
                               jnp.full((16,), jnp.sum(acc_s), jnp.float32),
                               mask=lane0)

        pltpu.sync_copy(nbuf.at[par], num_hbm.at[pl.ds(g * CHUNK, CHUNK)])
        pltpu.sync_copy(sbuf.at[par], sq_hbm.at[pl.ds(g * CHUNK, CHUNK)])
        return carry

    jax.lax.fori_loop(0, nt, chunk_body, 0)


@functools.cache
def _get_sc_score():
    return functools.partial(
        pl.kernel,
        out_type=(
            jax.ShapeDtypeStruct((ROWS_SC,), jnp.float32),
            jax.ShapeDtypeStruct((ROWS_SC,), jnp.float32),
        ),
        mesh=plsc.VectorSubcoreMesh(
            core_axis_name="c", subcore_axis_name="s", num_cores=NC,
            num_subcores=NS),
        scratch_types=(
            pltpu.VMEM((DIM_,), jnp.float32),
            pltpu.VMEM((2, CHUNK, DIM_), jnp.float32),
            pltpu.VMEM((2, CHUNK), jnp.float32),
            pltpu.VMEM((2, CHUNK), jnp.float32),
            pltpu.SemaphoreType.DMA((2,)),
        ),
        compiler_params=pltpu.CompilerParams(
            needs_layout_passes=False, use_tc_tiling_on_sc=True),
    )(_sc_score_body)


def _tc_score_kernel(w_ref, wv_ref, num_ref, sq_ref):
    tile = wv_ref[...]
    wcol = w_ref[...]
    num = jnp.dot(tile, wcol, preferred_element_type=jnp.float32)
    sq = jnp.dot(tile * tile, jnp.ones_like(wcol),
                 preferred_element_type=jnp.float32)
    num_ref[...] = num.reshape(1, 1, -1)
    sq_ref[...] = sq.reshape(1, 1, -1)


def _topk_kernel(w_ref, n_ref, s_ref, out_ref):
    wsq = jnp.sum(w_ref[...] * w_ref[...])
    s = n_ref[...] / (jnp.sqrt(s_ref[...] + 1e-9) * jnp.sqrt(wsq))
    rows = s.shape[0]
    row = jax.lax.broadcasted_iota(jnp.int32, (rows, 128), 0)
    col = jax.lax.broadcasted_iota(jnp.int32, (rows, 128), 1)
    flat = row * 128 + col
    big = jnp.int32(2147483647)
    for i in range(K):
        m = jnp.max(s)
        idx = jnp.min(jnp.where(s == m, flat, big))
        out_ref[i] = idx
        s = jnp.where(flat == idx, -jnp.inf, s)


def kernel(w, word2vec, k):
    vocab, dim = word2vec.shape
    wcol = w.reshape(dim, 1)

    sc_n, sc_sq = _get_sc_score()(w, word2vec)

    nb = F_TC // BLOCK
    tc_n, tc_sq = pl.pallas_call(
        _tc_score_kernel,
        grid=(nb,),
        in_specs=[
            pl.BlockSpec((dim, 1), lambda i: (0, 0)),
            pl.BlockSpec((BLOCK, dim), lambda i: (i, 0)),
        ],
        out_specs=[
            pl.BlockSpec((1, 1, BLOCK), lambda i: (i, 0, 0)),
            pl.BlockSpec((1, 1, BLOCK), lambda i: (i, 0, 0)),
        ],
        out_shape=[
            jax.ShapeDtypeStruct((nb, 1, BLOCK), jnp.float32),
            jax.ShapeDtypeStruct((nb, 1, BLOCK), jnp.float32),
        ],
    )(wcol, word2vec)

    num = jnp.concatenate([tc_n.reshape(-1), sc_n])
    sq = jnp.concatenate([tc_sq.reshape(-1), sc_sq])
    idx = pl.pallas_call(
        _topk_kernel,
        out_specs=pl.BlockSpec(memory_space=pltpu.SMEM),
        out_shape=jax.ShapeDtypeStruct((K,), jnp.int32),
    )(wcol, num.reshape(vocab // 128, 128), sq.reshape(vocab // 128, 128))
    return idx


# speedup vs baseline: 1.1314x; 1.1314x over previous
import functools

import jax
import jax.numpy as jnp
from jax.experimental import pallas as pl
from jax.experimental.pallas import tpu as pltpu
from jax.experimental.pallas import tpu_sc as plsc

K = 10
NC, NS = 2, 16
NW = NC * NS
CHUNK = 128
VOCAB_ = 400000
DIM_ = 300
VB = 4096
F_TC = 188416
ROWS_SC = VOCAB_ - F_TC


def _sc_score_body(w_hbm, wv_hbm, num_hbm, sq_hbm, wbuf, buf, nbuf, sbuf, sems):
    nchunks = ROWS_SC // CHUNK
    wid = jax.lax.axis_index("s") * NC + jax.lax.axis_index("c")
    nt = (nchunks - 1 - wid) // NW + 1

    pltpu.sync_copy(w_hbm, wbuf)
    lane = jax.lax.iota(jnp.int32, 16)
    m01 = jnp.where(lane >= 4, jnp.float32(1.0), jnp.float32(0.0))
    def _rb(v):
        c = v * jnp.float32(65537.0)
        return c - (c - v)

    wjs = [_rb(wbuf[pl.ds(16 * j, 16)]) for j in range(18)]
    wt = _rb(wbuf[pl.ds(284, 16)]) * m01

    def copy_in(t, par):
        g = wid + NW * t
        return pltpu.make_async_copy(
            wv_hbm.at[pl.ds(g * CHUNK, CHUNK), :],
            buf.at[par], sems.at[par])

    copy_in(0, 0).start()

    def chunk_body(t, carry):
        par = jax.lax.rem(t, 2)
        g = wid + NW * t
        copy_in(t, par).wait()

        @pl.when(t + 1 < nt)
        def _():
            copy_in(t + 1, 1 - par).start()

        @plsc.parallel_loop(0, CHUNK, unroll=2)
        def _row(r):
            x = buf[par, r, pl.ds(0, 16)]
            acc_n = _rb(x) * wjs[0]
            acc_s = x * x
            for j in range(1, 18):
                x = buf[par, r, pl.ds(16 * j, 16)]
                acc_n = acc_n + _rb(x) * wjs[j]
                acc_s = acc_s + x * x
            x = buf[par, r, pl.ds(284, 16)]
            acc_n = acc_n + _rb(x) * wt
            xm = x * m01
            acc_s = acc_s + xm * xm
            pvec = jnp.full((16,), par, jnp.int32)
            rvec = jnp.full((16,), r, jnp.int32)
            lane0 = lane == 0
            plsc.store_scatter(nbuf, [pvec, rvec],
                               jnp.full((16,), jnp.sum(acc_n), jnp.float32),
                               mask=lane0)
            plsc.store_scatter(sbuf, [pvec, rvec],
                               jnp.full((16,), jnp.sum(acc_s), jnp.float32),
                               mask=lane0)

        pltpu.sync_copy(nbuf.at[par], num_hbm.at[pl.ds(g * CHUNK, CHUNK)])
        pltpu.sync_copy(sbuf.at[par], sq_hbm.at[pl.ds(g * CHUNK, CHUNK)])
        return carry

    jax.lax.fori_loop(0, nt, chunk_body, 0)


@functools.cache
def _get_sc_score():
    return functools.partial(
        pl.kernel,
        out_type=(
            jax.ShapeDtypeStruct((ROWS_SC,), jnp.float32),
            jax.ShapeDtypeStruct((ROWS_SC,), jnp.float32),
        ),
        mesh=plsc.VectorSubcoreMesh(
            core_axis_name="c", subcore_axis_name="s", num_cores=NC,
            num_subcores=NS),
        scratch_types=(
            pltpu.VMEM((DIM_,), jnp.float32),
            pltpu.VMEM((2, CHUNK, DIM_), jnp.float32),
            pltpu.VMEM((2, CHUNK), jnp.float32),
            pltpu.VMEM((2, CHUNK), jnp.float32),
            pltpu.SemaphoreType.DMA((2,)),
        ),
        compiler_params=pltpu.CompilerParams(
            needs_layout_passes=False, use_tc_tiling_on_sc=True),
    )(_sc_score_body)


def _tc_score_kernel(w_ref, wvt_ref, num_ref, sq_ref):
    tile = wvt_ref[...]
    wrow = w_ref[...]
    num = jnp.dot(wrow.astype(jnp.bfloat16), tile.astype(jnp.bfloat16),
                  preferred_element_type=jnp.float32)
    sq = jnp.sum(tile * tile, axis=0, keepdims=True)
    num_ref[...] = num.reshape(1, 1, -1)
    sq_ref[...] = sq.reshape(1, 1, -1)


def _topk_kernel(w_ref, n_ref, s_ref, out_ref):
    wsq = jnp.sum(w_ref[...] * w_ref[...])
    s = n_ref[...] / (jnp.sqrt(s_ref[...] + 1e-9) * jnp.sqrt(wsq))
    rows = s.shape[0]
    row = jax.lax.broadcasted_iota(jnp.int32, (rows, 128), 0)
    col = jax.lax.broadcasted_iota(jnp.int32, (rows, 128), 1)
    flat = row * 128 + col
    big = jnp.int32(2147483647)
    for i in range(K):
        m = jnp.max(s)
        idx = jnp.min(jnp.where(s == m, flat, big))
        out_ref[i] = idx
        s = jnp.where(flat == idx, -jnp.inf, s)


def kernel(w, word2vec, k):
    vocab, dim = word2vec.shape
    wcol = w.reshape(dim, 1)

    sc_n, sc_sq = _get_sc_score()(w, jax.lax.slice(
        word2vec, (F_TC, 0), (vocab, dim)))

    wvt = word2vec.T
    nb = F_TC // VB
    tc_n, tc_sq = pl.pallas_call(
        _tc_score_kernel,
        grid=(nb,),
        in_specs=[
            pl.BlockSpec((1, dim), lambda i: (0, 0)),
            pl.BlockSpec((dim, VB), lambda i: (0, i)),
        ],
        out_specs=[
            pl.BlockSpec((1, 1, VB), lambda i: (i, 0, 0)),
            pl.BlockSpec((1, 1, VB), lambda i: (i, 0, 0)),
        ],
        out_shape=[
            jax.ShapeDtypeStruct((nb, 1, VB), jnp.float32),
            jax.ShapeDtypeStruct((nb, 1, VB), jnp.float32),
        ],
    )(w.reshape(1, dim), wvt)

    num = jnp.concatenate([tc_n.reshape(-1), sc_n])
    sq = jnp.concatenate([tc_sq.reshape(-1), sc_sq])
    idx = pl.pallas_call(
        _topk_kernel,
        out_specs=pl.BlockSpec(memory_space=pltpu.SMEM),
        out_shape=jax.ShapeDtypeStruct((K,), jnp.int32),
    )(wcol, num.reshape(vocab // 128, 128), sq.reshape(vocab // 128, 128))
    return idx


# speedup vs baseline: 3.8776x; 3.4273x over previous
import functools

import jax
import jax.numpy as jnp
from jax.experimental import pallas as pl
from jax.experimental.pallas import tpu as pltpu
from jax.experimental.pallas import tpu_sc as plsc

K = 10
NC, NS = 2, 16
NW = NC * NS
CHUNK = 128
VOCAB_ = 400000
DIM_ = 300
VB = 4096
F_TC = 245760
COLS_SC = VOCAB_ - F_TC
DMAIN = 296
DPAD = 304


def _rb(v):
    c = v * jnp.float32(65537.0)
    return c - (c - v)


def _sc_score_body(w_hbm, wvt_hbm, tail_hbm, num_hbm, sq_hbm, wbuf, wrbuf,
                   buf, nbuf, sbuf, sems, tsems):
    nchunks = COLS_SC // CHUNK
    wid = jax.lax.axis_index("s") * NC + jax.lax.axis_index("c")
    nt = (nchunks - 1 - wid) // NW + 1

    pltpu.sync_copy(w_hbm, wbuf)
    for j in range(19):
        base = 16 * j if j < 18 else 284
        wv = _rb(wbuf[pl.ds(base, 16)])
        for i in range(16):
            wrbuf[pl.ds(16 * (base + i), 16)] = jnp.full(
                (16,), wv[i], jnp.float32)
    for i in range(4):
        wrbuf[pl.ds(16 * (300 + i), 16)] = jnp.zeros((16,), jnp.float32)

    def copy_main(t, par):
        g = wid + NW * t
        return pltpu.make_async_copy(
            wvt_hbm.at[pl.ds(0, DMAIN), pl.ds(F_TC + g * CHUNK, CHUNK)],
            buf.at[par, pl.ds(0, DMAIN), :], sems.at[par])

    def copy_tail(t, par):
        g = wid + NW * t
        return pltpu.make_async_copy(
            tail_hbm.at[pl.ds(0, 8), pl.ds(F_TC + g * CHUNK, CHUNK)],
            buf.at[par, pl.ds(DMAIN, 8), :], tsems.at[par])

    copy_main(0, 0).start()
    copy_tail(0, 0).start()

    def chunk_body(t, carry):
        par = jax.lax.rem(t, 2)
        g = wid + NW * t
        copy_main(t, par).wait()
        copy_tail(t, par).wait()

        @pl.when(t + 1 < nt)
        def _():
            copy_main(t + 1, 1 - par).start()
            copy_tail(t + 1, 1 - par).start()

        for gi in range(CHUNK // 16):
            z = jnp.zeros((16,), jnp.float32)

            @plsc.parallel_loop(0, DPAD, unroll=4, carry=(z, z))
            def _d(d, acc):
                acc_n, acc_s = acc
                wdv = wrbuf[pl.ds(16 * d, 16)]
                x = buf[par, d, pl.ds(16 * gi, 16)]
                acc_n = acc_n + _rb(x) * wdv
                acc_s = acc_s + x * x
                return (acc_n, acc_s)

            acc_n, acc_s = _d
            nbuf[par, pl.ds(16 * gi, 16)] = acc_n
            sbuf[par, pl.ds(16 * gi, 16)] = acc_s

        pltpu.sync_copy(nbuf.at[par], num_hbm.at[pl.ds(g * CHUNK, CHUNK)])
        pltpu.sync_copy(sbuf.at[par], sq_hbm.at[pl.ds(g * CHUNK, CHUNK)])
        return carry

    jax.lax.fori_loop(0, nt, chunk_body, 0)


@functools.cache
def _get_sc_score():
    return functools.partial(
        pl.kernel,
        out_type=(
            jax.ShapeDtypeStruct((COLS_SC,), jnp.float32),
            jax.ShapeDtypeStruct((COLS_SC,), jnp.float32),
        ),
        mesh=plsc.VectorSubcoreMesh(
            core_axis_name="c", subcore_axis_name="s", num_cores=NC,
            num_subcores=NS),
        scratch_types=(
            pltpu.VMEM((DIM_,), jnp.float32),
            pltpu.VMEM((16 * DPAD,), jnp.float32),
            pltpu.VMEM((2, DPAD, CHUNK), jnp.float32),
            pltpu.VMEM((2, CHUNK), jnp.float32),
            pltpu.VMEM((2, CHUNK), jnp.float32),
            pltpu.SemaphoreType.DMA((2,)),
            pltpu.SemaphoreType.DMA((2,)),
        ),
        compiler_params=pltpu.CompilerParams(needs_layout_passes=False),
    )(_sc_score_body)


def _tc_score_kernel(w_ref, wvt_ref, num_ref, sq_ref):
    tile = wvt_ref[...]
    wrow = w_ref[...]
    num = jnp.dot(wrow.astype(jnp.bfloat16), tile.astype(jnp.bfloat16),
                  preferred_element_type=jnp.float32)
    sq = jnp.sum(tile * tile, axis=0, keepdims=True)
    num_ref[...] = num.reshape(1, 1, -1)
    sq_ref[...] = sq.reshape(1, 1, -1)


def _topk_kernel(w_ref, n_ref, s_ref, out_ref):
    wsq = jnp.sum(w_ref[...] * w_ref[...])
    s = n_ref[...] / (jnp.sqrt(s_ref[...] + 1e-9) * jnp.sqrt(wsq))
    rows = s.shape[0]
    row = jax.lax.broadcasted_iota(jnp.int32, (rows, 128), 0)
    col = jax.lax.broadcasted_iota(jnp.int32, (rows, 128), 1)
    flat = row * 128 + col
    big = jnp.int32(2147483647)
    for i in range(K):
        m = jnp.max(s)
        idx = jnp.min(jnp.where(s == m, flat, big))
        out_ref[i] = idx
        s = jnp.where(flat == idx, -jnp.inf, s)


def kernel(w, word2vec, k):
    vocab, dim = word2vec.shape
    wvt = word2vec.T
    tail8 = jnp.concatenate(
        [jax.lax.slice(wvt, (DMAIN, 0), (dim, vocab)),
         jnp.zeros((8 - (dim - DMAIN), vocab), jnp.float32)])

    sc_n, sc_sq = _get_sc_score()(w, wvt, tail8)

    nb = F_TC // VB
    tc_n, tc_sq = pl.pallas_call(
        _tc_score_kernel,
        grid=(nb,),
        in_specs=[
            pl.BlockSpec((1, dim), lambda i: (0, 0)),
            pl.BlockSpec((dim, VB), lambda i: (0, i)),
        ],
        out_specs=[
            pl.BlockSpec((1, 1, VB), lambda i: (i, 0, 0)),
            pl.BlockSpec((1, 1, VB), lambda i: (i, 0, 0)),
        ],
        out_shape=[
            jax.ShapeDtypeStruct((nb, 1, VB), jnp.float32),
            jax.ShapeDtypeStruct((nb, 1, VB), jnp.float32),
        ],
    )(w.reshape(1, dim), wvt)

    num = jnp.concatenate([tc_n.reshape(-1), sc_n])
    sq = jnp.concatenate([tc_sq.reshape(-1), sc_sq])
    wcol = w.reshape(dim, 1)
    idx = pl.pallas_call(
        _topk_kernel,
        out_specs=pl.BlockSpec(memory_space=pltpu.SMEM),
        out_shape=jax.ShapeDtypeStruct((K,), jnp.int32),
    )(wcol, num.reshape(vocab // 128, 128), sq.reshape(vocab // 128, 128))
    return idx
